# baseline (speedup 1.0000x reference)
import functools

import jax
import jax.numpy as jnp
from jax import lax
from jax.experimental import pallas as pl
from jax.experimental.pallas import tpu as pltpu
from jax.experimental.pallas import tpu_sc as plsc

NUM_ENT = 100000
NUM_REL = 1000
MARGIN = 10.0
B = 16384
D = 128

NC = 2
NS = 16
NW = NC * NS
EW = B // NW
CHUNK = 128
NCH = EW // CHUNK


def _sc_mesh():
    return plsc.VectorSubcoreMesh(core_axis_name="c", subcore_axis_name="s",
                                  num_cores=NC, num_subcores=NS)


def _make_gather(n_tabs, spec):
    nslot = len(spec)

    def body(ih, it, ir, *rest):
        tabs = rest[:n_tabs]
        out = rest[n_tabs]
        ihv, itv, irv, buf, sem0, sem1 = rest[n_tabs + 1:]
        sems = (sem0, sem1)
        idxvs = (ihv, itv, irv)

        wid = lax.axis_index("s") * NC + lax.axis_index("c")
        pltpu.sync_copy(ih.at[wid], ihv)
        pltpu.sync_copy(it.at[wid], itv)
        pltpu.sync_copy(ir.at[wid], irv)

        def chunk_body(c, carry):
            rowbase = wid * EW + c * CHUNK

            def start(k):
                ti, ii = spec[k]
                return pltpu.async_copy(tabs[ti].at[idxvs[ii].at[c]],
                                        buf.at[k % 2], sems[k % 2])

            cp = start(0)
            for k in range(1, nslot):
                cp_next = start(k)
                cp.wait()
                pltpu.sync_copy(buf.at[(k - 1) % 2],
                                out.at[k - 1, pl.ds(rowbase, CHUNK)])
                cp = cp_next
            cp.wait()
            pltpu.sync_copy(buf.at[(nslot - 1) % 2],
                            out.at[nslot - 1, pl.ds(rowbase, CHUNK)])
            return carry

        lax.fori_loop(0, NCH, chunk_body, 0)

    def run(ih, it, ir, tables):
        f = pl.kernel(
            body,
            out_type=jax.ShapeDtypeStruct((nslot, B, D), jnp.float32),
            mesh=_sc_mesh(),
            scratch_types=[
                pltpu.VMEM((NCH, CHUNK), jnp.int32),
                pltpu.VMEM((NCH, CHUNK), jnp.int32),
                pltpu.VMEM((NCH, CHUNK), jnp.int32),
                pltpu.VMEM((2, CHUNK, D), jnp.float32),
                pltpu.SemaphoreType.DMA,
                pltpu.SemaphoreType.DMA,
            ],
            compiler_params=pltpu.CompilerParams(use_tc_tiling_on_sc=True),
        )
        return f(ih, it, ir, *tables)

    return run


BT = 512


def _temb_body(t_ref, g_ref, o_ref):
    tt = t_ref[:]

    def S(k):
        return g_ref[k]

    o_ref[0] = S(2) * jnp.sin(S(0) * tt + S(1))
    o_ref[1] = S(5) * jnp.sin(S(3) * tt + S(4))


def _tc_temb(tvec, g):
    return pl.pallas_call(
        _temb_body,
        grid=(B // BT,),
        in_specs=[
            pl.BlockSpec((BT, 1), lambda i: (i, 0)),
            pl.BlockSpec((6, BT, D), lambda i: (0, i, 0)),
        ],
        out_specs=pl.BlockSpec((2, BT, D), lambda i: (0, i, 0)),
        out_shape=jax.ShapeDtypeStruct((2, B, D), jnp.float32),
    )(tvec, g)


def _combine_body(ge_ref, py_ref, pm_ref, pd_ref, o_ref):
    a_h = ge_ref[0]
    a_t = ge_ref[1]
    r = ge_ref[2]
    t_h = py_ref[0] + pm_ref[0] + pd_ref[0]
    t_t = py_ref[1] + pm_ref[1] + pd_ref[1]
    cr = jnp.cos(r)
    sr = jnp.sin(r)

    def part(h_re, h_im, t_re, t_im, c, s):
        re = h_re * c - h_im * s - t_re
        im = h_re * s + h_im * c - t_im
        return jnp.sum(jnp.sqrt(re * re + im * im), axis=1)

    H = 64
    tot = part(a_h[:, :H], a_t[:, H:], a_t[:, :H], a_h[:, H:],
               cr[:, :H], sr[:, :H])
    tot += part(t_h[:, :H], t_t[:, :H], t_t[:, H:], t_h[:, H:],
                cr[:, H:], sr[:, H:])
    o_ref[:] = MARGIN - tot


def _tc_combine(ge, py, pm, pd):
    p2 = pl.BlockSpec((2, BT, D), lambda i: (0, i, 0))
    return pl.pallas_call(
        _combine_body,
        grid=(B // BT,),
        in_specs=[pl.BlockSpec((3, BT, D), lambda i: (0, i, 0)), p2, p2, p2],
        out_specs=pl.BlockSpec((BT,), lambda i: (i,)),
        out_shape=jax.ShapeDtypeStruct((B,), jnp.float32),
    )(ge, py, pm, pd)


_GATHER_E = _make_gather(2, [(0, 0), (0, 1), (1, 2)])
_GATHER_P = _make_gather(3, [(0, 0), (1, 0), (2, 0), (0, 1), (1, 1), (2, 1)])

EB = 1024


def _pack_body(a_ref, b_ref, o_ref):
    o_ref[:, :64] = a_ref[:].T
    o_ref[:, 64:] = b_ref[:].T


def _tc_pack(at, bt):
    return pl.pallas_call(
        _pack_body,
        grid=((NUM_ENT + EB - 1) // EB,),
        in_specs=[
            pl.BlockSpec((64, EB), lambda i: (0, i)),
            pl.BlockSpec((64, EB), lambda i: (0, i)),
        ],
        out_specs=pl.BlockSpec((EB, D), lambda i: (i, 0)),
        out_shape=jax.ShapeDtypeStruct((NUM_ENT, D), jnp.float32),
    )(at, bt)


def kernel(heads, rels, tails, years, months, days, ent_embs_h, ent_embs_t,
           rel_embs, y_freq_h, y_freq_t, y_phi_h, y_phi_t, y_amps_h,
           y_amps_t, m_freq_h, m_freq_t, m_phi_h, m_phi_t, m_amps_h,
           m_amps_t, d_freq_h, d_freq_t, d_phi_h, d_phi_t, d_amps_h,
           d_amps_t):
    ih = heads.astype(jnp.int32).reshape(NW, NCH, CHUNK)
    it = tails.astype(jnp.int32).reshape(NW, NCH, CHUNK)
    ir = rels.astype(jnp.int32).reshape(NW, NCH, CHUNK)

    def pack(a, b):
        return _tc_pack(a.T, b.T)

    ge = _GATHER_E(ih, it, ir, (pack(ent_embs_h, ent_embs_t), rel_embs))
    gy = _GATHER_P(ih, it, ir, (pack(y_freq_h, y_freq_t),
                                pack(y_phi_h, y_phi_t),
                                pack(y_amps_h, y_amps_t)))
    gm = _GATHER_P(ih, it, ir, (pack(m_freq_h, m_freq_t),
                                pack(m_phi_h, m_phi_t),
                                pack(m_amps_h, m_amps_t)))
    gd = _GATHER_P(ih, it, ir, (pack(d_freq_h, d_freq_t),
                                pack(d_phi_h, d_phi_t),
                                pack(d_amps_h, d_amps_t)))
    py = _tc_temb(years.reshape(B, 1), gy)
    pm = _tc_temb(months.reshape(B, 1), gm)
    pd = _tc_temb(days.reshape(B, 1), gd)
    return _tc_combine(ge, py, pm, pd)


# speedup vs baseline: 1.4249x; 1.4249x over previous
import functools

import jax
import jax.numpy as jnp
from jax import lax
from jax.experimental import pallas as pl
from jax.experimental.pallas import tpu as pltpu
from jax.experimental.pallas import tpu_sc as plsc

NUM_ENT = 100000
NUM_REL = 1000
MARGIN = 10.0
B = 16384
D = 128

NC = 2
NS = 16
NW = NC * NS
EW = B // NW
CHUNK = 128
NCH = EW // CHUNK


def _sc_mesh():
    return plsc.VectorSubcoreMesh(core_axis_name="c", subcore_axis_name="s",
                                  num_cores=NC, num_subcores=NS)


def _make_gather(n_tabs, spec):
    nslot = len(spec)

    def body(ih, it, ir, *rest):
        tabs = rest[:n_tabs]
        out = rest[n_tabs]
        ihv, itv, irv, buf, sem0, sem1 = rest[n_tabs + 1:]
        sems = (sem0, sem1)
        idxvs = (ihv, itv, irv)

        wid = lax.axis_index("s") * NC + lax.axis_index("c")
        pltpu.sync_copy(ih.at[wid], ihv)
        pltpu.sync_copy(it.at[wid], itv)
        pltpu.sync_copy(ir.at[wid], irv)

        def chunk_body(c, carry):
            rowbase = wid * EW + c * CHUNK

            def start(k):
                ti, ii = spec[k]
                return pltpu.async_copy(tabs[ti].at[idxvs[ii].at[c]],
                                        buf.at[k % 2], sems[k % 2])

            cp = start(0)
            for k in range(1, nslot):
                cp_next = start(k)
                cp.wait()
                pltpu.sync_copy(buf.at[(k - 1) % 2],
                                out.at[k - 1, pl.ds(rowbase, CHUNK)])
                cp = cp_next
            cp.wait()
            pltpu.sync_copy(buf.at[(nslot - 1) % 2],
                            out.at[nslot - 1, pl.ds(rowbase, CHUNK)])
            return carry

        lax.fori_loop(0, NCH, chunk_body, 0)

    def run(ih, it, ir, tables):
        f = pl.kernel(
            body,
            out_type=jax.ShapeDtypeStruct((nslot, B, D), jnp.float32),
            mesh=_sc_mesh(),
            scratch_types=[
                pltpu.VMEM((NCH, CHUNK), jnp.int32),
                pltpu.VMEM((NCH, CHUNK), jnp.int32),
                pltpu.VMEM((NCH, CHUNK), jnp.int32),
                pltpu.VMEM((2, CHUNK, D), jnp.float32),
                pltpu.SemaphoreType.DMA,
                pltpu.SemaphoreType.DMA,
            ],
            compiler_params=pltpu.CompilerParams(use_tc_tiling_on_sc=True),
        )
        return f(ih, it, ir, *tables)

    return run


BT = 512


def _temb_body(t_ref, g_ref, o_ref):
    tt = t_ref[:]

    def S(k):
        return g_ref[k]

    o_ref[0] = S(2) * jnp.sin(S(0) * tt + S(1))
    o_ref[1] = S(5) * jnp.sin(S(3) * tt + S(4))


def _tc_temb(tvec, g):
    return pl.pallas_call(
        _temb_body,
        grid=(B // BT,),
        in_specs=[
            pl.BlockSpec((BT, 1), lambda i: (i, 0)),
            pl.BlockSpec((6, BT, D), lambda i: (0, i, 0)),
        ],
        out_specs=pl.BlockSpec((2, BT, D), lambda i: (0, i, 0)),
        out_shape=jax.ShapeDtypeStruct((2, B, D), jnp.float32),
    )(tvec, g)


def _combine_body(y_ref, m_ref, d_ref, ge_ref, gy_ref, gm_ref, gd_ref, o_ref):
    a_h = ge_ref[0]
    a_t = ge_ref[1]
    r = ge_ref[2]

    def term(g, t_ref, side):
        tt = t_ref[:]
        b = 3 * side
        return g[b + 2] * jnp.sin(g[b] * tt + g[b + 1])

    t_h = (term(gy_ref, y_ref, 0) + term(gm_ref, m_ref, 0)
           + term(gd_ref, d_ref, 0))
    t_t = (term(gy_ref, y_ref, 1) + term(gm_ref, m_ref, 1)
           + term(gd_ref, d_ref, 1))
    cr = jnp.cos(r)
    sr = jnp.sin(r)

    def part(h_re, h_im, t_re, t_im, c, s):
        re = h_re * c - h_im * s - t_re
        im = h_re * s + h_im * c - t_im
        return jnp.sum(jnp.sqrt(re * re + im * im), axis=1)

    H = 64
    tot = part(a_h[:, :H], a_t[:, H:], a_t[:, :H], a_h[:, H:],
               cr[:, :H], sr[:, :H])
    tot += part(t_h[:, :H], t_t[:, :H], t_t[:, H:], t_h[:, H:],
                cr[:, H:], sr[:, H:])
    o_ref[:] = MARGIN - tot


def _tc_combine(years, months, days, ge, gy, gm, gd):
    p6 = pl.BlockSpec((6, BT, D), lambda i: (0, i, 0))
    pt = pl.BlockSpec((BT, 1), lambda i: (i, 0))
    return pl.pallas_call(
        _combine_body,
        grid=(B // BT,),
        in_specs=[pt, pt, pt,
                  pl.BlockSpec((3, BT, D), lambda i: (0, i, 0)), p6, p6, p6],
        out_specs=pl.BlockSpec((BT,), lambda i: (i,)),
        out_shape=jax.ShapeDtypeStruct((B,), jnp.float32),
    )(years, months, days, ge, gy, gm, gd)


_GATHER_E = _make_gather(2, [(0, 0), (0, 1), (1, 2)])
_GATHER_P = _make_gather(3, [(0, 0), (1, 0), (2, 0), (0, 1), (1, 1), (2, 1)])

EB = 4096


def _pack_body(a_ref, b_ref, o_ref):
    o_ref[:, :64] = a_ref[:].T
    o_ref[:, 64:] = b_ref[:].T


def _tc_pack(at, bt):
    return pl.pallas_call(
        _pack_body,
        grid=((NUM_ENT + EB - 1) // EB,),
        in_specs=[
            pl.BlockSpec((64, EB), lambda i: (0, i)),
            pl.BlockSpec((64, EB), lambda i: (0, i)),
        ],
        out_specs=pl.BlockSpec((EB, D), lambda i: (i, 0)),
        out_shape=jax.ShapeDtypeStruct((NUM_ENT, D), jnp.float32),
    )(at, bt)


def kernel(heads, rels, tails, years, months, days, ent_embs_h, ent_embs_t,
           rel_embs, y_freq_h, y_freq_t, y_phi_h, y_phi_t, y_amps_h,
           y_amps_t, m_freq_h, m_freq_t, m_phi_h, m_phi_t, m_amps_h,
           m_amps_t, d_freq_h, d_freq_t, d_phi_h, d_phi_t, d_amps_h,
           d_amps_t):
    ih = heads.astype(jnp.int32).reshape(NW, NCH, CHUNK)
    it = tails.astype(jnp.int32).reshape(NW, NCH, CHUNK)
    ir = rels.astype(jnp.int32).reshape(NW, NCH, CHUNK)

    def pack(a, b):
        return _tc_pack(a.T, b.T)

    ge = _GATHER_E(ih, it, ir, (pack(ent_embs_h, ent_embs_t), rel_embs))
    gy = _GATHER_P(ih, it, ir, (pack(y_freq_h, y_freq_t),
                                pack(y_phi_h, y_phi_t),
                                pack(y_amps_h, y_amps_t)))
    gm = _GATHER_P(ih, it, ir, (pack(m_freq_h, m_freq_t),
                                pack(m_phi_h, m_phi_t),
                                pack(m_amps_h, m_amps_t)))
    gd = _GATHER_P(ih, it, ir, (pack(d_freq_h, d_freq_t),
                                pack(d_phi_h, d_phi_t),
                                pack(d_amps_h, d_amps_t)))
    return _tc_combine(years.reshape(B, 1), months.reshape(B, 1),
                       days.reshape(B, 1), ge, gy, gm, gd)


# speedup vs baseline: 1.7187x; 1.2062x over previous
import functools

import jax
import jax.numpy as jnp
from jax import lax
from jax.experimental import pallas as pl
from jax.experimental.pallas import tpu as pltpu
from jax.experimental.pallas import tpu_sc as plsc

NUM_ENT = 100000
NUM_REL = 1000
MARGIN = 10.0
B = 16384
D = 128

NC = 2
NS = 16
NW = NC * NS
EW = B // NW
CHUNK = 128
NCH = EW // CHUNK


def _sc_mesh():
    return plsc.VectorSubcoreMesh(core_axis_name="c", subcore_axis_name="s",
                                  num_cores=NC, num_subcores=NS)


def _make_gather(n_tabs, spec):
    nslot = len(spec)

    def body(ih, it, ir, *rest):
        tabs = rest[:n_tabs]
        out = rest[n_tabs]
        ihv, itv, irv, buf, sem0, sem1 = rest[n_tabs + 1:]
        sems = (sem0, sem1)
        idxvs = (ihv, itv, irv)

        wid = lax.axis_index("s") * NC + lax.axis_index("c")
        pltpu.sync_copy(ih.at[wid], ihv)
        pltpu.sync_copy(it.at[wid], itv)
        pltpu.sync_copy(ir.at[wid], irv)

        def chunk_body(c, carry):
            rowbase = wid * EW + c * CHUNK

            def start(k):
                ti, ii = spec[k]
                return pltpu.async_copy(tabs[ti].at[idxvs[ii].at[c]],
                                        buf.at[k % 2], sems[k % 2])

            cp = start(0)
            for k in range(1, nslot):
                cp_next = start(k)
                cp.wait()
                pltpu.sync_copy(buf.at[(k - 1) % 2],
                                out.at[k - 1, pl.ds(rowbase, CHUNK)])
                cp = cp_next
            cp.wait()
            pltpu.sync_copy(buf.at[(nslot - 1) % 2],
                            out.at[nslot - 1, pl.ds(rowbase, CHUNK)])
            return carry

        lax.fori_loop(0, NCH, chunk_body, 0)

    def run(ih, it, ir, tables):
        f = pl.kernel(
            body,
            out_type=jax.ShapeDtypeStruct((nslot, B, D), jnp.float32),
            mesh=_sc_mesh(),
            scratch_types=[
                pltpu.VMEM((NCH, CHUNK), jnp.int32),
                pltpu.VMEM((NCH, CHUNK), jnp.int32),
                pltpu.VMEM((NCH, CHUNK), jnp.int32),
                pltpu.VMEM((2, CHUNK, D), jnp.float32),
                pltpu.SemaphoreType.DMA,
                pltpu.SemaphoreType.DMA,
            ],
            compiler_params=pltpu.CompilerParams(use_tc_tiling_on_sc=True),
        )
        return f(ih, it, ir, *tables)

    return run


BT = 512

_MAGIC = 12582912.0
_INV2PI = 0.15915493667125702
_P2A = 6.28125
_P2B = 0.0019353071795864769
_S1 = 0.99998459
_S2 = -0.16663258
_S3 = 0.0083123829
_S4 = -0.00019316182
_S5 = 2.1732101e-06
_HALF_PI = 1.5707963267948966


def _psin(x):
    n = (x * _INV2PI + _MAGIC) - _MAGIC
    r = (x - n * _P2A) - n * _P2B
    r2 = r * r
    return r * (_S1 + r2 * (_S2 + r2 * (_S3 + r2 * (_S4 + r2 * _S5))))


def _pcos(x):
    return _psin(x + _HALF_PI)


def _temb_body(t_ref, g_ref, o_ref):
    tt = t_ref[:]

    def S(k):
        return g_ref[k]

    o_ref[0] = S(2) * jnp.sin(S(0) * tt + S(1))
    o_ref[1] = S(5) * jnp.sin(S(3) * tt + S(4))


def _tc_temb(tvec, g):
    return pl.pallas_call(
        _temb_body,
        grid=(B // BT,),
        in_specs=[
            pl.BlockSpec((BT, 1), lambda i: (i, 0)),
            pl.BlockSpec((6, BT, D), lambda i: (0, i, 0)),
        ],
        out_specs=pl.BlockSpec((2, BT, D), lambda i: (0, i, 0)),
        out_shape=jax.ShapeDtypeStruct((2, B, D), jnp.float32),
    )(tvec, g)


def _combine_body(y_ref, m_ref, d_ref, ge_ref, gy_ref, gm_ref, gd_ref, o_ref):
    a_h = ge_ref[0]
    a_t = ge_ref[1]
    r = ge_ref[2]

    def term(g, t_ref, side):
        tt = t_ref[:]
        b = 3 * side
        return g[b + 2] * _psin(g[b] * tt + g[b + 1])

    t_h = (term(gy_ref, y_ref, 0) + term(gm_ref, m_ref, 0)
           + term(gd_ref, d_ref, 0))
    t_t = (term(gy_ref, y_ref, 1) + term(gm_ref, m_ref, 1)
           + term(gd_ref, d_ref, 1))
    cr = _pcos(r)
    sr = _psin(r)

    def part(h_re, h_im, t_re, t_im, c, s):
        re = h_re * c - h_im * s - t_re
        im = h_re * s + h_im * c - t_im
        return jnp.sum(jnp.sqrt(re * re + im * im), axis=1)

    H = 64
    tot = part(a_h[:, :H], a_t[:, H:], a_t[:, :H], a_h[:, H:],
               cr[:, :H], sr[:, :H])
    tot += part(t_h[:, :H], t_t[:, :H], t_t[:, H:], t_h[:, H:],
                cr[:, H:], sr[:, H:])
    o_ref[:] = MARGIN - tot


def _tc_combine(years, months, days, ge, gy, gm, gd):
    p6 = pl.BlockSpec((6, BT, D), lambda i: (0, i, 0))
    pt = pl.BlockSpec((BT, 1), lambda i: (i, 0))
    return pl.pallas_call(
        _combine_body,
        grid=(B // BT,),
        in_specs=[pt, pt, pt,
                  pl.BlockSpec((3, BT, D), lambda i: (0, i, 0)), p6, p6, p6],
        out_specs=pl.BlockSpec((BT,), lambda i: (i,)),
        out_shape=jax.ShapeDtypeStruct((B,), jnp.float32),
    )(years, months, days, ge, gy, gm, gd)


_GATHER_E = _make_gather(2, [(0, 0), (0, 1), (1, 2)])
_GATHER_P = _make_gather(3, [(0, 0), (1, 0), (2, 0), (0, 1), (1, 1), (2, 1)])

EB = 4096


def _pack_body(a_ref, b_ref, o_ref):
    o_ref[:, :64] = a_ref[:].T
    o_ref[:, 64:] = b_ref[:].T


def _tc_pack(at, bt):
    return pl.pallas_call(
        _pack_body,
        grid=((NUM_ENT + EB - 1) // EB,),
        in_specs=[
            pl.BlockSpec((64, EB), lambda i: (0, i)),
            pl.BlockSpec((64, EB), lambda i: (0, i)),
        ],
        out_specs=pl.BlockSpec((EB, D), lambda i: (i, 0)),
        out_shape=jax.ShapeDtypeStruct((NUM_ENT, D), jnp.float32),
    )(at, bt)


def kernel(heads, rels, tails, years, months, days, ent_embs_h, ent_embs_t,
           rel_embs, y_freq_h, y_freq_t, y_phi_h, y_phi_t, y_amps_h,
           y_amps_t, m_freq_h, m_freq_t, m_phi_h, m_phi_t, m_amps_h,
           m_amps_t, d_freq_h, d_freq_t, d_phi_h, d_phi_t, d_amps_h,
           d_amps_t):
    ih = heads.astype(jnp.int32).reshape(NW, NCH, CHUNK)
    it = tails.astype(jnp.int32).reshape(NW, NCH, CHUNK)
    ir = rels.astype(jnp.int32).reshape(NW, NCH, CHUNK)

    def pack(a, b):
        return _tc_pack(a.T, b.T)

    ge = _GATHER_E(ih, it, ir, (pack(ent_embs_h, ent_embs_t), rel_embs))
    gy = _GATHER_P(ih, it, ir, (pack(y_freq_h, y_freq_t),
                                pack(y_phi_h, y_phi_t),
                                pack(y_amps_h, y_amps_t)))
    gm = _GATHER_P(ih, it, ir, (pack(m_freq_h, m_freq_t),
                                pack(m_phi_h, m_phi_t),
                                pack(m_amps_h, m_amps_t)))
    gd = _GATHER_P(ih, it, ir, (pack(d_freq_h, d_freq_t),
                                pack(d_phi_h, d_phi_t),
                                pack(d_amps_h, d_amps_t)))
    return _tc_combine(years.reshape(B, 1), months.reshape(B, 1),
                       days.reshape(B, 1), ge, gy, gm, gd)


# speedup vs baseline: 1.8241x; 1.0613x over previous
import functools

import jax
import jax.numpy as jnp
from jax import lax
from jax.experimental import pallas as pl
from jax.experimental.pallas import tpu as pltpu
from jax.experimental.pallas import tpu_sc as plsc

NUM_ENT = 100000
NUM_REL = 1000
MARGIN = 10.0
B = 16384
D = 128

NC = 2
NS = 16
NW = NC * NS
EW = B // NW
CHUNK = 128
NCH = EW // CHUNK


def _sc_mesh():
    return plsc.VectorSubcoreMesh(core_axis_name="c", subcore_axis_name="s",
                                  num_cores=NC, num_subcores=NS)


def _make_gather(n_tabs, spec):
    nslot = len(spec)

    def body(ih, it, ir, *rest):
        tabs = rest[:n_tabs]
        out = rest[n_tabs]
        ihv, itv, irv, buf, sem0, sem1 = rest[n_tabs + 1:]
        sems = (sem0, sem1)
        idxvs = (ihv, itv, irv)

        wid = lax.axis_index("s") * NC + lax.axis_index("c")
        pltpu.sync_copy(ih.at[wid], ihv)
        pltpu.sync_copy(it.at[wid], itv)
        pltpu.sync_copy(ir.at[wid], irv)

        def chunk_body(c, carry):
            rowbase = wid * EW + c * CHUNK

            def start(k):
                ti, ii = spec[k]
                return pltpu.async_copy(tabs[ti].at[idxvs[ii].at[c]],
                                        buf.at[k % 2], sems[k % 2])

            cp = start(0)
            for k in range(1, nslot):
                cp_next = start(k)
                cp.wait()
                pltpu.sync_copy(buf.at[(k - 1) % 2],
                                out.at[k - 1, pl.ds(rowbase, CHUNK)])
                cp = cp_next
            cp.wait()
            pltpu.sync_copy(buf.at[(nslot - 1) % 2],
                            out.at[nslot - 1, pl.ds(rowbase, CHUNK)])
            return carry

        lax.fori_loop(0, NCH, chunk_body, 0)

    def run(ih, it, ir, tables):
        f = pl.kernel(
            body,
            out_type=jax.ShapeDtypeStruct((nslot, B, D), jnp.float32),
            mesh=_sc_mesh(),
            scratch_types=[
                pltpu.VMEM((NCH, CHUNK), jnp.int32),
                pltpu.VMEM((NCH, CHUNK), jnp.int32),
                pltpu.VMEM((NCH, CHUNK), jnp.int32),
                pltpu.VMEM((2, CHUNK, D), jnp.float32),
                pltpu.SemaphoreType.DMA,
                pltpu.SemaphoreType.DMA,
            ],
            compiler_params=pltpu.CompilerParams(use_tc_tiling_on_sc=True),
        )
        return f(ih, it, ir, *tables)

    return run


BT = 512

_MAGIC = 12582912.0
_INV2PI = 0.15915493667125702
_P2A = 6.28125
_P2B = 0.0019353071795864769
_S1 = 0.99998459
_S2 = -0.16663258
_S3 = 0.0083123829
_S4 = -0.00019316182
_S5 = 2.1732101e-06
_HALF_PI = 1.5707963267948966


def _psin(x):
    n = (x * _INV2PI + _MAGIC) - _MAGIC
    r = (x - n * _P2A) - n * _P2B
    r2 = r * r
    return r * (_S1 + r2 * (_S2 + r2 * (_S3 + r2 * (_S4 + r2 * _S5))))


def _pcos(x):
    return _psin(x + _HALF_PI)


def _temb_body(t_ref, g_ref, o_ref):
    tt = t_ref[:]

    def S(k):
        return g_ref[k]

    o_ref[0] = S(2) * jnp.sin(S(0) * tt + S(1))
    o_ref[1] = S(5) * jnp.sin(S(3) * tt + S(4))


def _tc_temb(tvec, g):
    return pl.pallas_call(
        _temb_body,
        grid=(B // BT,),
        in_specs=[
            pl.BlockSpec((BT, 1), lambda i: (i, 0)),
            pl.BlockSpec((6, BT, D), lambda i: (0, i, 0)),
        ],
        out_specs=pl.BlockSpec((2, BT, D), lambda i: (0, i, 0)),
        out_shape=jax.ShapeDtypeStruct((2, B, D), jnp.float32),
    )(tvec, g)


def _combine_body(y_ref, m_ref, d_ref, ge_ref, gy_ref, gm_ref, gd_ref, o_ref):
    a_h = ge_ref[0]
    a_t = ge_ref[1]
    r = ge_ref[2]

    def term(g, t_ref, side):
        tt = t_ref[:]
        b = 3 * side
        return g[b + 2] * _psin(g[b] * tt + g[b + 1])

    t_h = (term(gy_ref, y_ref, 0) + term(gm_ref, m_ref, 0)
           + term(gd_ref, d_ref, 0))
    t_t = (term(gy_ref, y_ref, 1) + term(gm_ref, m_ref, 1)
           + term(gd_ref, d_ref, 1))
    cr = _pcos(r)
    sr = _psin(r)

    def part(h_re, h_im, t_re, t_im, c, s):
        re = h_re * c - h_im * s - t_re
        im = h_re * s + h_im * c - t_im
        return jnp.sum(jnp.sqrt(re * re + im * im), axis=1)

    H = 64
    tot = part(a_h[:, :H], a_t[:, H:], a_t[:, :H], a_h[:, H:],
               cr[:, :H], sr[:, :H])
    tot += part(t_h[:, :H], t_t[:, :H], t_t[:, H:], t_h[:, H:],
                cr[:, H:], sr[:, H:])
    o_ref[:] = MARGIN - tot


def _tc_combine(years, months, days, ge, gy, gm, gd):
    p6 = pl.BlockSpec((6, BT, D), lambda i: (0, i, 0))
    pt = pl.BlockSpec((BT, 1), lambda i: (i, 0))
    return pl.pallas_call(
        _combine_body,
        grid=(B // BT,),
        in_specs=[pt, pt, pt,
                  pl.BlockSpec((3, BT, D), lambda i: (0, i, 0)), p6, p6, p6],
        out_specs=pl.BlockSpec((BT,), lambda i: (i,)),
        out_shape=jax.ShapeDtypeStruct((B,), jnp.float32),
    )(years, months, days, ge, gy, gm, gd)


_GATHER_E = _make_gather(2, [(0, 0), (0, 1), (1, 2)])
_GATHER_P = _make_gather(3, [(0, 0), (1, 0), (2, 0), (0, 1), (1, 1), (2, 1)])

EB = 8192


def _pack_body(a_ref, b_ref, o_ref):
    o_ref[:, :64] = a_ref[:].T
    o_ref[:, 64:] = b_ref[:].T


def _tc_pack(at, bt):
    return pl.pallas_call(
        _pack_body,
        grid=((NUM_ENT + EB - 1) // EB,),
        in_specs=[
            pl.BlockSpec((64, EB), lambda i: (0, i)),
            pl.BlockSpec((64, EB), lambda i: (0, i)),
        ],
        out_specs=pl.BlockSpec((EB, D), lambda i: (i, 0)),
        out_shape=jax.ShapeDtypeStruct((NUM_ENT, D), jnp.float32),
    )(at, bt)


def kernel(heads, rels, tails, years, months, days, ent_embs_h, ent_embs_t,
           rel_embs, y_freq_h, y_freq_t, y_phi_h, y_phi_t, y_amps_h,
           y_amps_t, m_freq_h, m_freq_t, m_phi_h, m_phi_t, m_amps_h,
           m_amps_t, d_freq_h, d_freq_t, d_phi_h, d_phi_t, d_amps_h,
           d_amps_t):
    ih = heads.astype(jnp.int32).reshape(NW, NCH, CHUNK)
    it = tails.astype(jnp.int32).reshape(NW, NCH, CHUNK)
    ir = rels.astype(jnp.int32).reshape(NW, NCH, CHUNK)

    def pack(a, b):
        return _tc_pack(a.T, b.T)

    ge = _GATHER_E(ih, it, ir, (pack(ent_embs_h, ent_embs_t), rel_embs))
    gy = _GATHER_P(ih, it, ir, (pack(y_freq_h, y_freq_t),
                                pack(y_phi_h, y_phi_t),
                                pack(y_amps_h, y_amps_t)))
    gm = _GATHER_P(ih, it, ir, (pack(m_freq_h, m_freq_t),
                                pack(m_phi_h, m_phi_t),
                                pack(m_amps_h, m_amps_t)))
    gd = _GATHER_P(ih, it, ir, (pack(d_freq_h, d_freq_t),
                                pack(d_phi_h, d_phi_t),
                                pack(d_amps_h, d_amps_t)))
    return _tc_combine(years.reshape(B, 1), months.reshape(B, 1),
                       days.reshape(B, 1), ge, gy, gm, gd)
